# baseline (speedup 1.0000x reference)
import functools

import jax
import jax.numpy as jnp
from jax import lax
from jax.experimental import pallas as pl
from jax.experimental.pallas import tpu as pltpu
from jax.experimental.pallas import tpu_sc as plsc
from jax.experimental.layout import Format, Layout, with_layout_constraint

_B = 16384
_D = 64
_NEG = 20
_ITEMS = 1000000

_NC = 2
_NS = 16
_NW = _NC * _NS
_L = 16
_G = _D // _L

_BPW = _B // _NW
_CH = 64
_NCH = _BPW // _CH
_IDX_CH = 128


def _sc_body(ii_hbm, pi_hbm, ni_hbm, ev_hbm, eu_hbm,
             pos_out, neg_out,
             idx_in, idx_pos, idx_neg, v_buf, p_buf, n_buf,
             pos_buf, neg_buf, sem):
    w = lax.axis_index("s") * _NC + lax.axis_index("c")
    base = w * _BPW

    pltpu.sync_copy(ii_hbm.at[pl.ds(base, _BPW)], idx_in)
    pltpu.sync_copy(pi_hbm.at[pl.ds(base, _BPW)], idx_pos)
    pltpu.sync_copy(ni_hbm.at[pl.ds(base * _NEG, _BPW * _NEG)], idx_neg)

    def chunk_body(c, _):
        cb = pl.multiple_of(c * _CH, _CH)
        cps = [
            pltpu.async_copy(ev_hbm.at[idx_in.at[pl.ds(cb, _CH)]], v_buf, sem),
            pltpu.async_copy(eu_hbm.at[idx_pos.at[pl.ds(cb, _CH)]], p_buf, sem),
        ]
        nbase = pl.multiple_of(c * (_CH * _NEG), _CH * _NEG)
        for k in range(_CH * _NEG // _IDX_CH):
            cps.append(pltpu.async_copy(
                eu_hbm.at[idx_neg.at[pl.ds(nbase + k * _IDX_CH, _IDX_CH)]],
                n_buf.at[pl.ds(k * _IDX_CH, _IDX_CH)], sem))
        for cp in cps:
            cp.wait()

        def b_body(b, _):
            nb = b * _NEG
            ob = cb + b
            pos_acc = None
            neg_acc = None
            def up(x):
                return plsc.unpack(x, format=plsc.PackFormat.INTERLEAVED,
                                   preferred_element_type=jnp.float32)

            for h in range(_D // 32):
                sl = pl.ds(h * 32, 32)
                va, vb = up(v_buf[b, sl])
                ta, tb = up(n_buf[nb, sl])
                for j in range(1, _NEG):
                    na, nbv = up(n_buf[nb + j, sl])
                    ta = ta + na
                    tb = tb + nbv
                pa, pb = up(p_buf[b, sl])
                pacc = va * pa + vb * pb
                nacc = va * ta + vb * tb
                pos_acc = pacc if h == 0 else pos_acc + pacc
                neg_acc = nacc if h == 0 else neg_acc + nacc
            pos_buf[ob, :] = pos_acc
            neg_buf[ob, :] = neg_acc
            return 0

        lax.fori_loop(0, _CH, b_body, 0)
        return 0

    lax.fori_loop(0, _NCH, chunk_body, 0)
    pltpu.sync_copy(pos_buf, pos_out.at[pl.ds(base, _BPW)])
    pltpu.sync_copy(neg_buf, neg_out.at[pl.ds(base, _BPW)])


@functools.cache
def _sc_scores():
  return pl.kernel(
    _sc_body,
    out_type=(
        jax.ShapeDtypeStruct((_B, _L), jnp.float32),
        jax.ShapeDtypeStruct((_B, _L), jnp.float32),
    ),
    mesh=plsc.VectorSubcoreMesh(core_axis_name="c", subcore_axis_name="s",
                                num_cores=_NC, num_subcores=_NS),
    compiler_params=pltpu.CompilerParams(use_tc_tiling_on_sc=False,
                                         needs_layout_passes=False),
    scratch_types=[
        pltpu.VMEM((_BPW,), jnp.int32),
        pltpu.VMEM((_BPW,), jnp.int32),
        pltpu.VMEM((_BPW * _NEG,), jnp.int32),
        pltpu.VMEM((_CH, _D), jnp.bfloat16),
        pltpu.VMEM((_CH, _D), jnp.bfloat16),
        pltpu.VMEM((_CH * _NEG, _D), jnp.bfloat16),
        pltpu.VMEM((_BPW, _L), jnp.float32),
        pltpu.VMEM((_BPW, _L), jnp.float32),
        pltpu.SemaphoreType.DMA,
    ],
  )


def _vgather_body(ii_hbm, ev_hbm, vout_hbm, idx_v, sem):
    w = lax.axis_index("s") * _NC + lax.axis_index("c")
    base = w * _BPW
    pltpu.sync_copy(ii_hbm.at[pl.ds(base, _BPW)], idx_v)
    iota = lax.iota(jnp.int32, 16)

    def g_body(g, _):
        gb = pl.multiple_of(g * 16, 16)
        ivec = idx_v[pl.ds(gb, 16)]
        cps = []
        for l in range(16):
            s = lax.reduce_max(jnp.where(iota == l, ivec, 0), axes=(0,))
            cps.append(pltpu.async_copy(
                ev_hbm.at[pl.ds(s, 1)],
                vout_hbm.at[pl.ds(base + gb + l, 1)], sem))
        for cp in cps:
            cp.wait()
        return 0

    lax.fori_loop(0, _BPW // 16, g_body, 0)


@functools.cache
def _sc_vgather():
  return pl.kernel(
    _vgather_body,
    out_type=jax.ShapeDtypeStruct((_B, _D), jnp.float32),
    mesh=plsc.VectorSubcoreMesh(core_axis_name="c", subcore_axis_name="s",
                                num_cores=_NC, num_subcores=_NS),
    compiler_params=pltpu.CompilerParams(use_tc_tiling_on_sc=True,
                                         needs_layout_passes=False),
    scratch_types=[
        pltpu.VMEM((_BPW,), jnp.int32),
        pltpu.SemaphoreType.DMA,
    ],
  )


def _log_sigmoid(x):
    return jnp.minimum(x, 0.0) - jnp.log1p(jnp.exp(-jnp.abs(x)))


def _loss_body(pos_ref, neg_ref, out_ref):
    pos = jnp.sum(pos_ref[...], axis=1, keepdims=True)
    neg = -jnp.sum(neg_ref[...], axis=1, keepdims=True)
    loss = _log_sigmoid(pos) + _log_sigmoid(neg)
    out_ref[...] = -jnp.sum(loss, axis=(0, 1), keepdims=True) / _B


_tc_loss = pl.pallas_call(
    _loss_body,
    out_shape=jax.ShapeDtypeStruct((1, 1), jnp.float32),
)


def kernel(input_items, pos_items, neg_items, embedding_v, embedding_u):
    ii = input_items.reshape(_B)
    pi = pos_items.reshape(_B)
    ni = neg_items.reshape(_B * _NEG)
    fmt = Layout(major_to_minor=(0, 1), tiling=((16,),))

    def _prep(t):
        t = t.astype(jnp.bfloat16).reshape(_ITEMS * _D // 128, 128)
        t = with_layout_constraint(t, fmt)
        return with_layout_constraint(t.reshape(_ITEMS, _D), fmt)

    ev = _prep(embedding_v)
    eu = _prep(embedding_u)
    v_rows = _sc_vgather()(ii, embedding_v)
    pos_part, neg_part = _sc_scores()(ii, pi, ni, ev, eu)
    return (_tc_loss(pos_part, neg_part) + 0.0 * v_rows[0, 0]).reshape(())


# speedup vs baseline: 1.6623x; 1.6623x over previous
import functools

import jax
import jax.numpy as jnp
from jax import lax
from jax.experimental import pallas as pl
from jax.experimental.pallas import tpu as pltpu
from jax.experimental.pallas import tpu_sc as plsc
from jax.experimental.layout import Format, Layout, with_layout_constraint

_B = 16384
_D = 64
_NEG = 20
_ITEMS = 1000000

_NC = 2
_NS = 16
_NW = _NC * _NS
_L = 16
_G = _D // _L

_BPW = _B // _NW
_CH = 64
_NCH = _BPW // _CH
_IDX_CH = 128


def _sc_body(ii_hbm, pi_hbm, ni_hbm, ev_hbm, eu_hbm,
             pos_out, neg_out,
             idx_in, idx_pos, idx_neg, v_buf, p_buf, n_buf,
             pos_buf, neg_buf, sem):
    w = lax.axis_index("s") * _NC + lax.axis_index("c")
    base = w * _BPW

    pltpu.sync_copy(ii_hbm.at[pl.ds(base, _BPW)], idx_in)
    pltpu.sync_copy(pi_hbm.at[pl.ds(base, _BPW)], idx_pos)
    pltpu.sync_copy(ni_hbm.at[pl.ds(base * _NEG, _BPW * _NEG)], idx_neg)

    def chunk_body(c, _):
        cb = pl.multiple_of(c * _CH, _CH)
        cps = [
            pltpu.async_copy(ev_hbm.at[idx_in.at[pl.ds(cb, _CH)]], v_buf, sem),
            pltpu.async_copy(eu_hbm.at[idx_pos.at[pl.ds(cb, _CH)]], p_buf, sem),
        ]
        nbase = pl.multiple_of(c * (_CH * _NEG), _CH * _NEG)
        for k in range(_CH * _NEG // _IDX_CH):
            cps.append(pltpu.async_copy(
                eu_hbm.at[idx_neg.at[pl.ds(nbase + k * _IDX_CH, _IDX_CH)]],
                n_buf.at[pl.ds(k * _IDX_CH, _IDX_CH)], sem))
        for cp in cps:
            cp.wait()

        def b_body(b, _):
            nb = b * _NEG
            ob = cb + b
            pos_acc = None
            neg_acc = None
            for g in range(_G):
                sl = pl.ds(g * _L, _L)
                vv = v_buf[b, sl]
                t = n_buf[nb, sl]
                for j in range(1, _NEG):
                    t = t + n_buf[nb + j, sl]
                pa = vv * p_buf[b, sl]
                na = vv * t
                pos_acc = pa if g == 0 else pos_acc + pa
                neg_acc = na if g == 0 else neg_acc + na
            pos_buf[ob, :] = pos_acc
            neg_buf[ob, :] = neg_acc
            return 0

        lax.fori_loop(0, _CH, b_body, 0)
        return 0

    lax.fori_loop(0, _NCH, chunk_body, 0)
    pltpu.sync_copy(pos_buf, pos_out.at[pl.ds(base, _BPW)])
    pltpu.sync_copy(neg_buf, neg_out.at[pl.ds(base, _BPW)])


@functools.cache
def _sc_scores():
  return pl.kernel(
    _sc_body,
    out_type=(
        jax.ShapeDtypeStruct((_B, _L), jnp.float32),
        jax.ShapeDtypeStruct((_B, _L), jnp.float32),
    ),
    mesh=plsc.VectorSubcoreMesh(core_axis_name="c", subcore_axis_name="s",
                                num_cores=_NC, num_subcores=_NS),
    compiler_params=pltpu.CompilerParams(use_tc_tiling_on_sc=False),
    scratch_types=[
        pltpu.VMEM((_BPW,), jnp.int32),
        pltpu.VMEM((_BPW,), jnp.int32),
        pltpu.VMEM((_BPW * _NEG,), jnp.int32),
        pltpu.VMEM((_CH, _D), jnp.float32),
        pltpu.VMEM((_CH, _D), jnp.float32),
        pltpu.VMEM((_CH * _NEG, _D), jnp.float32),
        pltpu.VMEM((_BPW, _L), jnp.float32),
        pltpu.VMEM((_BPW, _L), jnp.float32),
        pltpu.SemaphoreType.DMA,
    ],
  )


def _log_sigmoid(x):
    return jnp.minimum(x, 0.0) - jnp.log1p(jnp.exp(-jnp.abs(x)))


def _loss_body(pos_ref, neg_ref, out_ref):
    pos = jnp.sum(pos_ref[...], axis=1, keepdims=True)
    neg = -jnp.sum(neg_ref[...], axis=1, keepdims=True)
    loss = _log_sigmoid(pos) + _log_sigmoid(neg)
    out_ref[...] = -jnp.sum(loss, axis=(0, 1), keepdims=True) / _B


_tc_loss = pl.pallas_call(
    _loss_body,
    out_shape=jax.ShapeDtypeStruct((1, 1), jnp.float32),
)


def kernel(input_items, pos_items, neg_items, embedding_v, embedding_u):
    ii = input_items.reshape(_B)
    pi = pos_items.reshape(_B)
    ni = neg_items.reshape(_B * _NEG)
    fmt = Layout(major_to_minor=(0, 1), tiling=((8,),))
    ev = with_layout_constraint(embedding_v, fmt)
    eu = with_layout_constraint(embedding_u, fmt)
    pos_part, neg_part = _sc_scores()(ii, pi, ni, ev, eu)
    return _tc_loss(pos_part, neg_part).reshape(())


# speedup vs baseline: 1.7090x; 1.0280x over previous
import functools

import jax
import jax.numpy as jnp
from jax import lax
from jax.experimental import pallas as pl
from jax.experimental.pallas import tpu as pltpu
from jax.experimental.pallas import tpu_sc as plsc
from jax.experimental.layout import Format, Layout, with_layout_constraint

_B = 16384
_D = 64
_NEG = 20
_ITEMS = 1000000

_NC = 2
_NS = 16
_NW = _NC * _NS
_L = 16
_G = _D // _L

_BPW = _B // _NW
_CH = 32
_NCH = _BPW // _CH
_IDX_CH = 128


def _sc_body(ii_hbm, pi_hbm, ni_hbm, ev_hbm, eu_hbm,
             pos_out, neg_out,
             idx_in, idx_pos, idx_neg,
             v_buf0, p_buf0, n_buf0, v_buf1, p_buf1, n_buf1,
             pos_buf, neg_buf, sem0, sem1):
    w = lax.axis_index("s") * _NC + lax.axis_index("c")
    base = w * _BPW

    pltpu.sync_copy(ii_hbm.at[pl.ds(base, _BPW)], idx_in)
    pltpu.sync_copy(pi_hbm.at[pl.ds(base, _BPW)], idx_pos)
    pltpu.sync_copy(ni_hbm.at[pl.ds(base * _NEG, _BPW * _NEG)], idx_neg)

    bufs = ((v_buf0, p_buf0, n_buf0, sem0), (v_buf1, p_buf1, n_buf1, sem1))

    def copies(c, par):
        v_buf, p_buf, n_buf, sem = bufs[par]
        cb = pl.multiple_of(c * _CH, _CH)
        cps = [
            pltpu.make_async_copy(ev_hbm.at[idx_in.at[pl.ds(cb, _CH)]],
                                  v_buf, sem),
            pltpu.make_async_copy(eu_hbm.at[idx_pos.at[pl.ds(cb, _CH)]],
                                  p_buf, sem),
        ]
        nbase = pl.multiple_of(c * (_CH * _NEG), _CH * _NEG)
        for k in range(_CH * _NEG // _IDX_CH):
            cps.append(pltpu.make_async_copy(
                eu_hbm.at[idx_neg.at[pl.ds(nbase + k * _IDX_CH, _IDX_CH)]],
                n_buf.at[pl.ds(k * _IDX_CH, _IDX_CH)], sem))
        return cps

    def issue(c, par):
        for cp in copies(c, par):
            cp.start()

    def drain(c, par):
        for cp in copies(c, par):
            cp.wait()

    def compute(c, par):
        v_buf, p_buf, n_buf, _ = bufs[par]
        cb = pl.multiple_of(c * _CH, _CH)

        def b_body(b, _):
            nb = b * _NEG
            ob = cb + b
            pos_acc = None
            neg_acc = None
            for g in range(_G):
                sl = pl.ds(g * _L, _L)
                vv = v_buf[b, sl]
                t = n_buf[nb, sl]
                for j in range(1, _NEG):
                    t = t + n_buf[nb + j, sl]
                pa = vv * p_buf[b, sl]
                na = vv * t
                pos_acc = pa if g == 0 else pos_acc + pa
                neg_acc = na if g == 0 else neg_acc + na
            pos_buf[ob, :] = pos_acc
            neg_buf[ob, :] = neg_acc
            return 0

        lax.fori_loop(0, _CH, b_body, 0)

    issue(0, 0)

    def pair_body(s, _):
        c0 = s * 2
        c1 = c0 + 1
        drain(c0, 0)
        issue(c1, 1)
        compute(c0, 0)
        drain(c1, 1)

        @pl.when(c0 + 2 < _NCH)
        def _():
            issue(c0 + 2, 0)

        compute(c1, 1)
        return 0

    lax.fori_loop(0, _NCH // 2, pair_body, 0)
    pltpu.sync_copy(pos_buf, pos_out.at[pl.ds(base, _BPW)])
    pltpu.sync_copy(neg_buf, neg_out.at[pl.ds(base, _BPW)])


@functools.cache
def _sc_scores():
  return pl.kernel(
    _sc_body,
    out_type=(
        jax.ShapeDtypeStruct((_B, _L), jnp.float32),
        jax.ShapeDtypeStruct((_B, _L), jnp.float32),
    ),
    mesh=plsc.VectorSubcoreMesh(core_axis_name="c", subcore_axis_name="s",
                                num_cores=_NC, num_subcores=_NS),
    compiler_params=pltpu.CompilerParams(use_tc_tiling_on_sc=False),
    scratch_types=[
        pltpu.VMEM((_BPW,), jnp.int32),
        pltpu.VMEM((_BPW,), jnp.int32),
        pltpu.VMEM((_BPW * _NEG,), jnp.int32),
        pltpu.VMEM((_CH, _D), jnp.float32),
        pltpu.VMEM((_CH, _D), jnp.float32),
        pltpu.VMEM((_CH * _NEG, _D), jnp.float32),
        pltpu.VMEM((_CH, _D), jnp.float32),
        pltpu.VMEM((_CH, _D), jnp.float32),
        pltpu.VMEM((_CH * _NEG, _D), jnp.float32),
        pltpu.VMEM((_BPW, _L), jnp.float32),
        pltpu.VMEM((_BPW, _L), jnp.float32),
        pltpu.SemaphoreType.DMA,
        pltpu.SemaphoreType.DMA,
    ],
  )


def _log_sigmoid(x):
    return jnp.minimum(x, 0.0) - jnp.log1p(jnp.exp(-jnp.abs(x)))


def _loss_body(pos_ref, neg_ref, out_ref):
    pos = jnp.sum(pos_ref[...], axis=1, keepdims=True)
    neg = -jnp.sum(neg_ref[...], axis=1, keepdims=True)
    loss = _log_sigmoid(pos) + _log_sigmoid(neg)
    out_ref[...] = -jnp.sum(loss, axis=(0, 1), keepdims=True) / _B


_tc_loss = pl.pallas_call(
    _loss_body,
    out_shape=jax.ShapeDtypeStruct((1, 1), jnp.float32),
)


def kernel(input_items, pos_items, neg_items, embedding_v, embedding_u):
    ii = input_items.reshape(_B)
    pi = pos_items.reshape(_B)
    ni = neg_items.reshape(_B * _NEG)
    fmt = Layout(major_to_minor=(0, 1), tiling=((8,),))
    ev = with_layout_constraint(embedding_v, fmt)
    eu = with_layout_constraint(embedding_u, fmt)
    pos_part, neg_part = _sc_scores()(ii, pi, ni, ev, eu)
    return _tc_loss(pos_part, neg_part).reshape(())
